# Initial kernel scaffold; baseline (speedup 1.0000x reference)
import jax
import jax.numpy as jnp
from jax.experimental import pallas as pl


def kernel(hidden_states, weight):
    raise NotImplementedError("write your pallas kernel here")


# speedup vs baseline: 2.7333x; 2.7333x over previous
import jax
import jax.numpy as jnp
from jax.experimental import pallas as pl
from jax.experimental.pallas import tpu as pltpu

B_, S_, H_, E_, K_ = 4, 8192, 1024, 64, 8
ALPHA_ = 0.1
TOK_BLK = 1024
AUX_SCALE = E_ * ALPHA_ / (B_ * float(S_) * float(S_) * K_)


def _gate_kernel(x_ref, w_ref, idx_ref, gate_ref, aux_ref, acc_ref, cnt_ref):
    step = pl.program_id(0)

    @pl.when(step == 0)
    def _init():
        acc_ref[...] = jnp.zeros_like(acc_ref)
        cnt_ref[...] = jnp.zeros_like(cnt_ref)

    x = x_ref[...]
    w = w_ref[...]
    logits = jax.lax.dot_general(
        x, w, (((1,), (1,)), ((), ())), preferred_element_type=jnp.float32
    )
    m = jnp.max(logits, axis=-1, keepdims=True)
    ex = jnp.exp(logits - m)
    scores = ex / jnp.sum(ex, axis=-1, keepdims=True)

    iota = jax.lax.broadcasted_iota(jnp.int32, scores.shape, 1)
    s = scores
    idx_cols = []
    val_cols = []
    cnt_blk = jnp.zeros((1, E_), jnp.float32)
    for _ in range(K_):
        vmax = jnp.max(s, axis=-1, keepdims=True)
        imax = jnp.min(jnp.where(s == vmax, iota, E_), axis=-1, keepdims=True)
        onehot = iota == imax
        cnt_blk = cnt_blk + jnp.sum(
            onehot.astype(jnp.float32), axis=0, keepdims=True
        )
        s = jnp.where(onehot, -1.0, s)
        idx_cols.append(imax)
        val_cols.append(vmax)

    vals = jnp.concatenate(val_cols, axis=-1)
    denom = jnp.sum(vals, axis=-1, keepdims=True) + 1e-20
    gate_ref[...] = vals / denom
    idx_ref[...] = jnp.concatenate(idx_cols, axis=-1).astype(jnp.int32)

    b = step // (S_ // TOK_BLK)
    ssum = jnp.sum(scores, axis=0, keepdims=True)
    acc_ref[pl.ds(b, 1), :] += ssum
    cnt_ref[pl.ds(b, 1), :] += cnt_blk

    @pl.when(step == pl.num_programs(0) - 1)
    def _finalize():
        aux_ref[...] = jnp.sum(
            acc_ref[...] * cnt_ref[...], keepdims=True
        ).reshape(1, 1) * AUX_SCALE


def kernel(hidden_states, weight):
    n = B_ * S_
    x = hidden_states.reshape(n, H_)
    grid = n // TOK_BLK
    idx, gate, aux = pl.pallas_call(
        _gate_kernel,
        grid=(grid,),
        in_specs=[
            pl.BlockSpec((TOK_BLK, H_), lambda i: (i, 0)),
            pl.BlockSpec((E_, H_), lambda i: (0, 0)),
        ],
        out_specs=[
            pl.BlockSpec((TOK_BLK, K_), lambda i: (i, 0)),
            pl.BlockSpec((TOK_BLK, K_), lambda i: (i, 0)),
            pl.BlockSpec((1, 1), lambda i: (0, 0)),
        ],
        out_shape=[
            jax.ShapeDtypeStruct((n, K_), jnp.int32),
            jax.ShapeDtypeStruct((n, K_), jnp.float32),
            jax.ShapeDtypeStruct((1, 1), jnp.float32),
        ],
        scratch_shapes=[
            pltpu.VMEM((B_, E_), jnp.float32),
            pltpu.VMEM((B_, E_), jnp.float32),
        ],
    )(x, weight)
    return (idx, gate, aux.reshape(()))


# speedup vs baseline: 5.0202x; 1.8367x over previous
import jax
import jax.numpy as jnp
from jax.experimental import pallas as pl
from jax.experimental.pallas import tpu as pltpu

B_, S_, H_, E_, K_ = 4, 8192, 1024, 64, 8
ALPHA_ = 0.1
TOK_BLK = 1024
AUX_SCALE = E_ * ALPHA_ / (B_ * float(S_) * float(S_) * K_)


def _gate_kernel(x_ref, w_ref, idx_ref, gate_ref, aux_ref, acc_ref, cnt_ref):
    step = pl.program_id(0)

    @pl.when(step == 0)
    def _init():
        acc_ref[...] = jnp.zeros_like(acc_ref)
        cnt_ref[...] = jnp.zeros_like(cnt_ref)

    x = x_ref[...]
    w = w_ref[...]
    logits = jax.lax.dot_general(
        w, x, (((1,), (1,)), ((), ())), preferred_element_type=jnp.float32
    )
    m = jnp.max(logits, axis=0, keepdims=True)
    ex = jnp.exp(logits - m)
    scores = ex / jnp.sum(ex, axis=0, keepdims=True)

    iota = jax.lax.broadcasted_iota(jnp.int32, scores.shape, 0).astype(
        jnp.float32
    )
    s = scores
    idx_rows = []
    val_rows = []
    oh_sum = jnp.zeros_like(scores)
    for _ in range(K_):
        vmax = jnp.max(s, axis=0, keepdims=True)
        imax = jnp.min(jnp.where(s == vmax, iota, 64.0), axis=0, keepdims=True)
        onehot = iota == imax
        oh_sum = oh_sum + onehot.astype(jnp.float32)
        s = jnp.where(onehot, -1.0, s)
        idx_rows.append(imax)
        val_rows.append(vmax)

    vals = jnp.concatenate(val_rows, axis=0)
    denom = jnp.sum(vals, axis=0, keepdims=True) + 1e-20
    gate_ref[...] = (vals / denom).T
    idx_ref[...] = jnp.concatenate(idx_rows, axis=0).T.astype(jnp.int32)

    b = step // (S_ // TOK_BLK)
    bmask = (
        jax.lax.broadcasted_iota(jnp.int32, (E_, B_), 1) == b
    ).astype(jnp.float32)
    acc_ref[...] += jnp.sum(scores, axis=1, keepdims=True) * bmask
    cnt_ref[...] += jnp.sum(oh_sum, axis=1, keepdims=True) * bmask

    @pl.when(step == pl.num_programs(0) - 1)
    def _finalize():
        aux_ref[...] = jnp.sum(
            acc_ref[...] * cnt_ref[...], keepdims=True
        ).reshape(1, 1) * AUX_SCALE


def kernel(hidden_states, weight):
    n = B_ * S_
    x = hidden_states.reshape(n, H_)
    grid = n // TOK_BLK
    idx, gate, aux = pl.pallas_call(
        _gate_kernel,
        grid=(grid,),
        in_specs=[
            pl.BlockSpec((TOK_BLK, H_), lambda i: (i, 0)),
            pl.BlockSpec((E_, H_), lambda i: (0, 0)),
        ],
        out_specs=[
            pl.BlockSpec((TOK_BLK, K_), lambda i: (i, 0)),
            pl.BlockSpec((TOK_BLK, K_), lambda i: (i, 0)),
            pl.BlockSpec((1, 1), lambda i: (0, 0)),
        ],
        out_shape=[
            jax.ShapeDtypeStruct((n, K_), jnp.int32),
            jax.ShapeDtypeStruct((n, K_), jnp.float32),
            jax.ShapeDtypeStruct((1, 1), jnp.float32),
        ],
        scratch_shapes=[
            pltpu.VMEM((E_, B_), jnp.float32),
            pltpu.VMEM((E_, B_), jnp.float32),
        ],
    )(x, weight)
    return (idx, gate, aux.reshape(()))


# speedup vs baseline: 5.7363x; 1.1426x over previous
import jax
import jax.numpy as jnp
from jax.experimental import pallas as pl
from jax.experimental.pallas import tpu as pltpu

B_, S_, H_, E_, K_ = 4, 8192, 1024, 64, 8
ALPHA_ = 0.1
TOK_BLK = 2048
AUX_SCALE = E_ * ALPHA_ / (B_ * float(S_) * float(S_) * K_)


def _gate_kernel(x_ref, w_ref, idx_ref, gate_ref, aux_ref, acc_ref, cnt_ref):
    step = pl.program_id(0)

    @pl.when(step == 0)
    def _init():
        acc_ref[...] = jnp.zeros_like(acc_ref)
        cnt_ref[...] = jnp.zeros_like(cnt_ref)

    x = x_ref[...]
    w = w_ref[...]
    logits = jax.lax.dot_general(
        w, x, (((1,), (1,)), ((), ())), preferred_element_type=jnp.float32
    )
    m = jnp.max(logits, axis=0, keepdims=True)
    ex = jnp.exp(logits - m)
    z = jnp.sum(ex, axis=0, keepdims=True)
    scores = ex / z

    iota = jax.lax.broadcasted_iota(jnp.int32, scores.shape, 0).astype(
        jnp.float32
    )
    s = scores
    idx_rows = []
    val_rows = []
    for k in range(K_):
        if k == 0:
            vmax = 1.0 / z
        else:
            vmax = jnp.max(s, axis=0, keepdims=True)
        cand = jnp.where(s == vmax, iota, 64.0)
        imax = jnp.min(cand, axis=0, keepdims=True)
        onehot = cand == imax
        s = jnp.where(onehot, -1.0, s)
        idx_rows.append(imax)
        val_rows.append(vmax)

    vals = jnp.concatenate(val_rows, axis=0)
    denom = jnp.sum(vals, axis=0, keepdims=True) + 1e-20
    gate_ref[...] = (vals / denom).T
    idx_ref[...] = jnp.concatenate(idx_rows, axis=0).T.astype(jnp.int32)

    b = step // (S_ // TOK_BLK)
    bmask = (
        jax.lax.broadcasted_iota(jnp.int32, (E_, B_), 1) == b
    ).astype(jnp.float32)
    chosen = jnp.where(s < 0.0, 1.0, 0.0)
    acc_ref[...] += jnp.sum(scores, axis=1, keepdims=True) * bmask
    cnt_ref[...] += jnp.sum(chosen, axis=1, keepdims=True) * bmask

    @pl.when(step == pl.num_programs(0) - 1)
    def _finalize():
        aux_ref[...] = jnp.sum(
            acc_ref[...] * cnt_ref[...], keepdims=True
        ).reshape(1, 1) * AUX_SCALE


def kernel(hidden_states, weight):
    n = B_ * S_
    x = hidden_states.reshape(n, H_)
    grid = n // TOK_BLK
    idx, gate, aux = pl.pallas_call(
        _gate_kernel,
        grid=(grid,),
        in_specs=[
            pl.BlockSpec((TOK_BLK, H_), lambda i: (i, 0)),
            pl.BlockSpec((E_, H_), lambda i: (0, 0)),
        ],
        out_specs=[
            pl.BlockSpec((TOK_BLK, K_), lambda i: (i, 0)),
            pl.BlockSpec((TOK_BLK, K_), lambda i: (i, 0)),
            pl.BlockSpec((1, 1), lambda i: (0, 0)),
        ],
        out_shape=[
            jax.ShapeDtypeStruct((n, K_), jnp.int32),
            jax.ShapeDtypeStruct((n, K_), jnp.float32),
            jax.ShapeDtypeStruct((1, 1), jnp.float32),
        ],
        scratch_shapes=[
            pltpu.VMEM((E_, B_), jnp.float32),
            pltpu.VMEM((E_, B_), jnp.float32),
        ],
    )(x, weight)
    return (idx, gate, aux.reshape(()))


# speedup vs baseline: 5.9899x; 1.0442x over previous
import jax
import jax.numpy as jnp
from jax.experimental import pallas as pl
from jax.experimental.pallas import tpu as pltpu

B_, S_, H_, E_, K_ = 4, 8192, 1024, 64, 8
ALPHA_ = 0.1
TOK_BLK = 4096
AUX_SCALE = E_ * ALPHA_ / (B_ * float(S_) * float(S_) * K_)


def _gate_kernel(x_ref, w_ref, idx_ref, gate_ref, aux_ref, acc_ref, cnt_ref):
    step = pl.program_id(0)

    @pl.when(step == 0)
    def _init():
        acc_ref[...] = jnp.zeros_like(acc_ref)
        cnt_ref[...] = jnp.zeros_like(cnt_ref)

    x = x_ref[...]
    w = w_ref[...]
    logits = jax.lax.dot_general(
        w, x, (((1,), (1,)), ((), ())), preferred_element_type=jnp.float32
    )
    m = jnp.max(logits, axis=0, keepdims=True)
    ex = jnp.exp(logits - m)
    z = jnp.sum(ex, axis=0, keepdims=True)
    scores = ex / z

    iota = jax.lax.broadcasted_iota(jnp.int32, scores.shape, 0).astype(
        jnp.float32
    )
    s = scores
    idx_rows = []
    val_rows = []
    for k in range(K_):
        if k == 0:
            vmax = 1.0 / z
        else:
            vmax = jnp.max(s, axis=0, keepdims=True)
        cand = jnp.where(s == vmax, iota, 64.0)
        imax = jnp.min(cand, axis=0, keepdims=True)
        onehot = cand == imax
        s = jnp.where(onehot, -1.0, s)
        idx_rows.append(imax)
        val_rows.append(vmax)

    vals = jnp.concatenate(val_rows, axis=0)
    denom = jnp.sum(vals, axis=0, keepdims=True) + 1e-20
    gate_ref[...] = (vals / denom).T
    idx_ref[...] = jnp.concatenate(idx_rows, axis=0).T.astype(jnp.int32)

    b = step // (S_ // TOK_BLK)
    bmask = (
        jax.lax.broadcasted_iota(jnp.int32, (E_, B_), 1) == b
    ).astype(jnp.float32)
    chosen = jnp.where(s < 0.0, 1.0, 0.0)
    acc_ref[...] += jnp.sum(scores, axis=1, keepdims=True) * bmask
    cnt_ref[...] += jnp.sum(chosen, axis=1, keepdims=True) * bmask

    @pl.when(step == pl.num_programs(0) - 1)
    def _finalize():
        aux_ref[...] = jnp.sum(
            acc_ref[...] * cnt_ref[...], keepdims=True
        ).reshape(1, 1) * AUX_SCALE


def kernel(hidden_states, weight):
    n = B_ * S_
    x = hidden_states.reshape(n, H_)
    grid = n // TOK_BLK
    idx, gate, aux = pl.pallas_call(
        _gate_kernel,
        grid=(grid,),
        in_specs=[
            pl.BlockSpec((TOK_BLK, H_), lambda i: (i, 0)),
            pl.BlockSpec((E_, H_), lambda i: (0, 0)),
        ],
        out_specs=[
            pl.BlockSpec((TOK_BLK, K_), lambda i: (i, 0)),
            pl.BlockSpec((TOK_BLK, K_), lambda i: (i, 0)),
            pl.BlockSpec((1, 1), lambda i: (0, 0)),
        ],
        out_shape=[
            jax.ShapeDtypeStruct((n, K_), jnp.int32),
            jax.ShapeDtypeStruct((n, K_), jnp.float32),
            jax.ShapeDtypeStruct((1, 1), jnp.float32),
        ],
        scratch_shapes=[
            pltpu.VMEM((E_, B_), jnp.float32),
            pltpu.VMEM((E_, B_), jnp.float32),
        ],
    )(x, weight)
    return (idx, gate, aux.reshape(()))


# speedup vs baseline: 9.5768x; 1.5988x over previous
import jax
import jax.numpy as jnp
from jax.experimental import pallas as pl
from jax.experimental.pallas import tpu as pltpu

B_, S_, H_, E_, K_ = 4, 8192, 1024, 64, 8
ALPHA_ = 0.1
TOK_BLK = 4096
AUX_SCALE = E_ * ALPHA_ / (B_ * float(S_) * float(S_) * K_)


def _gate_kernel(x_ref, w_ref, idx_ref, gate_ref, aux_ref, acc_ref, cnt_ref):
    step = pl.program_id(0)

    @pl.when(step == 0)
    def _init():
        acc_ref[...] = jnp.zeros_like(acc_ref)
        cnt_ref[...] = jnp.zeros_like(cnt_ref)

    x = x_ref[...]
    w = w_ref[...]
    logits = jax.lax.dot_general(
        w, x, (((1,), (1,)), ((), ())), preferred_element_type=jnp.float32
    )
    m = jnp.max(logits, axis=0, keepdims=True)
    ex = jnp.exp(logits - m)
    z = jnp.sum(ex, axis=0, keepdims=True)
    scores = ex / z

    iota = jax.lax.broadcasted_iota(jnp.int32, scores.shape, 0).astype(
        jnp.float32
    )
    s = scores
    idx_rows = []
    val_rows = []
    for k in range(K_):
        if k == 0:
            vmax = 1.0 / z
        else:
            vmax = jnp.max(s, axis=0, keepdims=True)
        cand = jnp.where(s == vmax, iota, 64.0)
        imax = jnp.min(cand, axis=0, keepdims=True)
        onehot = cand == imax
        s = jnp.where(onehot, -1.0, s)
        idx_rows.append(imax)
        val_rows.append(vmax)

    vals = jnp.concatenate(val_rows, axis=0)
    denom = jnp.sum(vals, axis=0, keepdims=True) + 1e-20
    gate_ref[...] = vals / denom
    idx_ref[...] = jnp.concatenate(idx_rows, axis=0).astype(jnp.int32)

    b = step // (S_ // TOK_BLK)
    bmask = (
        jax.lax.broadcasted_iota(jnp.int32, (E_, B_), 1) == b
    ).astype(jnp.float32)
    chosen = jnp.where(s < 0.0, 1.0, 0.0)
    acc_ref[...] += jnp.sum(scores, axis=1, keepdims=True) * bmask
    cnt_ref[...] += jnp.sum(chosen, axis=1, keepdims=True) * bmask

    @pl.when(step == pl.num_programs(0) - 1)
    def _finalize():
        aux_ref[...] = jnp.sum(
            acc_ref[...] * cnt_ref[...], keepdims=True
        ).reshape(1, 1) * AUX_SCALE


def kernel(hidden_states, weight):
    n = B_ * S_
    x = hidden_states.reshape(n, H_)
    grid = n // TOK_BLK
    idx8, gate8, aux = pl.pallas_call(
        _gate_kernel,
        grid=(grid,),
        in_specs=[
            pl.BlockSpec((TOK_BLK, H_), lambda i: (i, 0)),
            pl.BlockSpec((E_, H_), lambda i: (0, 0)),
        ],
        out_specs=[
            pl.BlockSpec((K_, TOK_BLK), lambda i: (0, i)),
            pl.BlockSpec((K_, TOK_BLK), lambda i: (0, i)),
            pl.BlockSpec((1, 1), lambda i: (0, 0)),
        ],
        out_shape=[
            jax.ShapeDtypeStruct((K_, n), jnp.int32),
            jax.ShapeDtypeStruct((K_, n), jnp.float32),
            jax.ShapeDtypeStruct((1, 1), jnp.float32),
        ],
        scratch_shapes=[
            pltpu.VMEM((E_, B_), jnp.float32),
            pltpu.VMEM((E_, B_), jnp.float32),
        ],
    )(x, weight)
    return (idx8.T, gate8.T, aux.reshape(()))
